# baseline (speedup 1.0000x reference)
import functools
import math
import jax
import jax.numpy as jnp
from jax import lax
from jax.experimental import pallas as pl
from jax.experimental.pallas import tpu as pltpu
from jax.experimental.pallas import tpu_sc as plsc

S, D, H, E, K, HID = 2048, 768, 12, 8, 2, 3072
DH = D // H
BM = 1024
BQ = 512
BMG = 256
P = S * K + E * BMG
NBLK = P // BMG
NC = 2
NS = 16
NW = NC * NS
TOK_W = S // NW


def _ln(x, g, b):
    m = jnp.mean(x, axis=-1, keepdims=True)
    xc = x - m
    v = jnp.mean(xc * xc, axis=-1, keepdims=True)
    return xc * jax.lax.rsqrt(v + 1e-6) * g + b


def _bf(x):
    return x.astype(jnp.bfloat16)


def _ln_qkv_body(x_ref, g_ref, b_ref, wq_ref, wk_ref, wv_ref,
                 q_ref, k_ref, v_ref):
    h = _bf(_ln(x_ref[...], g_ref[...], b_ref[...]))
    q = jnp.dot(h, wq_ref[...], preferred_element_type=jnp.float32)
    q_ref[...] = _bf(q * (1.0 / math.sqrt(DH)))
    k_ref[...] = _bf(jnp.dot(h, wk_ref[...],
                             preferred_element_type=jnp.float32))
    v_ref[...] = _bf(jnp.dot(h, wv_ref[...],
                             preferred_element_type=jnp.float32))


def _attn_body(q_ref, k_ref, v_ref, o_ref):
    q2 = q_ref[...]
    k2 = k_ref[...]
    v2 = v_ref[...]
    for j in range(4):
        q = q2[:, j * DH:(j + 1) * DH]
        k = k2[:, j * DH:(j + 1) * DH]
        v = v2[:, j * DH:(j + 1) * DH]
        s = jax.lax.dot_general(q, k, (((1,), (1,)), ((), ())),
                                preferred_element_type=jnp.float32)
        p = jnp.exp(s)
        den = jnp.sum(p, axis=-1, keepdims=True)
        o = jnp.dot(_bf(p), v, preferred_element_type=jnp.float32)
        o_ref[:, j * DH:(j + 1) * DH] = _bf(o / den)


def _post_body(a_ref, wo_ref, bo_ref, x_ref, g_ref, b_ref, wg_ref,
               x2_ref, h2_ref, lg_ref):
    o = jnp.dot(a_ref[...], wo_ref[...],
                preferred_element_type=jnp.float32)
    x2 = o + bo_ref[...] + x_ref[...]
    x2_ref[...] = x2
    h2 = _ln(x2, g_ref[...], b_ref[...])
    h2_ref[...] = h2
    lg_ref[...] = jnp.dot(h2, wg_ref[...], preferred_element_type=jnp.float32)


def _route_body(lg_ref, gw0_ref, gw1_ref, d0_ref, d1_ref, be_ref, bi_ref,
                na_ref):
    logits = lg_ref[...]
    iota = lax.broadcasted_iota(jnp.int32, (S, E), 1)
    v1 = jnp.max(logits, -1, keepdims=True)
    i1 = jnp.min(jnp.where(logits == v1, iota, E), -1, keepdims=True)
    l2 = jnp.where(iota == i1, -jnp.inf, logits)
    v2 = jnp.max(l2, -1, keepdims=True)
    i2 = jnp.min(jnp.where(l2 == v2, iota, E), -1, keepdims=True)
    r = jnp.exp(v2 - v1)
    gw0_ref[...] = 1.0 / (1.0 + r)
    gw1_ref[...] = r / (1.0 + r)

    oh0 = (iota == i1).astype(jnp.float32)
    oh1 = (iota == i2).astype(jnp.float32)
    soh = oh0 + oh1
    CH = 128
    a_io = lax.broadcasted_iota(jnp.int32, (CH, CH), 0)
    b_io = lax.broadcasted_iota(jnp.int32, (CH, CH), 1)
    tril = (a_io > b_io).astype(jnp.float32)
    run = jnp.zeros((1, E), jnp.float32)
    parts = []
    for c in range(S // CH):
        ch = soh[c * CH:(c + 1) * CH, :]
        parts.append(jnp.dot(tril, ch, preferred_element_type=jnp.float32)
                     + run)
        run = run + jnp.sum(ch, axis=0, keepdims=True)
    excl = jnp.concatenate(parts, axis=0)
    counts = run
    padded = jnp.ceil(counts / BMG) * BMG
    eu = lax.broadcasted_iota(jnp.int32, (E, E), 0)
    ev = lax.broadcasted_iota(jnp.int32, (E, E), 1)
    sut = (eu < ev).astype(jnp.float32)
    pstart = jnp.dot(padded, sut,
                     preferred_element_type=jnp.float32)
    pos = pstart + excl
    d0_ref[...] = jnp.sum(oh0 * pos, 1, keepdims=True).astype(jnp.int32)
    d1_ref[...] = jnp.sum(oh1 * pos, 1, keepdims=True).astype(jnp.int32)

    total = jnp.sum(padded, 1, keepdims=True)
    na = (total / BMG).astype(jnp.int32)
    na_ref[...] = na
    biota = lax.broadcasted_iota(jnp.int32, (1, 128), 1)
    bi_ref[...] = jnp.minimum(biota, na - 1)
    bstart = (biota * BMG).astype(jnp.float32)
    bexp = jnp.zeros((1, 128), jnp.int32)
    last_e = jnp.zeros((1, 1), jnp.float32)
    for e in range(E):
        ps_e = pstart[:, e:e + 1]
        pd_e = padded[:, e:e + 1]
        cond = (bstart >= ps_e) & (bstart < ps_e + pd_e)
        bexp = bexp + jnp.where(cond, e, 0)
        last_e = jnp.where(pd_e > 0, float(e), last_e)
    bexp = jnp.where(bstart >= total, last_e.astype(jnp.int32), bexp)
    be_ref[...] = bexp


def _gmm_body(be_ref, bi_ref, na_ref, xs_ref, w1_ref, b1_ref, w2_ref, b2_ref,
              out_ref):
    b = pl.program_id(0)

    @pl.when(b < na_ref[0])
    def _():
        mid = jnp.dot(_bf(xs_ref[...]), _bf(w1_ref[0]),
                      preferred_element_type=jnp.float32) + b1_ref[0]
        mid = _bf(jax.nn.gelu(mid))
        out_ref[...] = jnp.dot(mid, _bf(w2_ref[0]),
                               preferred_element_type=jnp.float32) + b2_ref[0]


def _combine_body(e0_ref, e1_ref, gw0_ref, gw1_ref, x2_ref, y_ref):
    y_ref[...] = (gw0_ref[...] * e0_ref[...] + gw1_ref[...] * e1_ref[...]
                  + x2_ref[...])


def _sc_dispatch_call(h2, d0, d1):
    mesh = plsc.VectorSubcoreMesh(core_axis_name="c", subcore_axis_name="s")

    @functools.partial(
        pl.kernel, mesh=mesh,
        out_type=jax.ShapeDtypeStruct((P, D), jnp.float32),
        scratch_types=[
            pltpu.VMEM((TOK_W,), jnp.int32),
            pltpu.VMEM((TOK_W,), jnp.int32),
            pltpu.VMEM((TOK_W, D), jnp.float32),
            pltpu.SemaphoreType.DMA,
            pltpu.SemaphoreType.DMA,
        ],
    )
    def k(h2_hbm, d0_hbm, d1_hbm, xs_hbm, i0_v, i1_v, rows_v, sem0, sem1):
        wid = lax.axis_index("s") * NC + lax.axis_index("c")
        base = wid * TOK_W
        pltpu.sync_copy(d0_hbm.at[pl.ds(base, TOK_W)], i0_v)
        pltpu.sync_copy(d1_hbm.at[pl.ds(base, TOK_W)], i1_v)
        pltpu.sync_copy(h2_hbm.at[pl.ds(base, TOK_W)], rows_v)
        c0 = pltpu.async_copy(rows_v, xs_hbm.at[i0_v], sem0)
        c1 = pltpu.async_copy(rows_v, xs_hbm.at[i1_v], sem1)
        c0.wait()
        c1.wait()

    return k(h2, d0, d1)


def _sc_gather_call(eout, d0, d1):
    mesh = plsc.VectorSubcoreMesh(core_axis_name="c", subcore_axis_name="s")

    @functools.partial(
        pl.kernel, mesh=mesh,
        out_type=[jax.ShapeDtypeStruct((S, D), jnp.float32),
                  jax.ShapeDtypeStruct((S, D), jnp.float32)],
        scratch_types=[
            pltpu.VMEM((TOK_W,), jnp.int32),
            pltpu.VMEM((TOK_W,), jnp.int32),
            pltpu.VMEM((TOK_W, D), jnp.float32),
            pltpu.VMEM((TOK_W, D), jnp.float32),
            pltpu.SemaphoreType.DMA,
            pltpu.SemaphoreType.DMA,
        ],
    )
    def k(eo_hbm, d0_hbm, d1_hbm, e0_hbm, e1_hbm,
          i0_v, i1_v, r0_v, r1_v, sem0, sem1):
        wid = lax.axis_index("s") * NC + lax.axis_index("c")
        base = wid * TOK_W
        pltpu.sync_copy(d0_hbm.at[pl.ds(base, TOK_W)], i0_v)
        pltpu.sync_copy(d1_hbm.at[pl.ds(base, TOK_W)], i1_v)
        c0 = pltpu.async_copy(eo_hbm.at[i0_v], r0_v, sem0)
        c1 = pltpu.async_copy(eo_hbm.at[i1_v], r1_v, sem1)
        c0.wait()
        c1.wait()
        pltpu.sync_copy(r0_v, e0_hbm.at[pl.ds(base, TOK_W)])
        pltpu.sync_copy(r1_v, e1_hbm.at[pl.ds(base, TOK_W)])

    return k(eout, d0, d1)


def kernel(x, Wq, Wk, Wv, Wo, bo, ln1_g, ln1_b, ln2_g, ln2_b,
           w_gate, W1, b1, W2, b2):
    xs = x.reshape(S, D)
    g1 = ln1_g.reshape(1, D)
    b1_ = ln1_b.reshape(1, D)
    g2 = ln2_g.reshape(1, D)
    b2_ = ln2_b.reshape(1, D)
    bo_ = bo.reshape(1, D)

    full = lambda *shape: pl.BlockSpec(shape, lambda *_: (0,) * len(shape))
    tok = pl.BlockSpec((BM, D), lambda i: (i, 0))
    f32 = jnp.float32
    bf16 = jnp.bfloat16

    q, k, v = pl.pallas_call(
        _ln_qkv_body,
        grid=(S // BM,),
        in_specs=[tok, full(1, D), full(1, D),
                  full(D, D), full(D, D), full(D, D)],
        out_specs=[tok, tok, tok],
        out_shape=[jax.ShapeDtypeStruct((S, D), bf16)] * 3,
    )(xs, g1, b1_, Wq.astype(bf16), Wk.astype(bf16), Wv.astype(bf16))

    attn = pl.pallas_call(
        _attn_body,
        grid=(H // 4, S // BQ),
        in_specs=[pl.BlockSpec((BQ, 4 * DH), lambda h, i: (i, h)),
                  pl.BlockSpec((S, 4 * DH), lambda h, i: (0, h)),
                  pl.BlockSpec((S, 4 * DH), lambda h, i: (0, h))],
        out_specs=pl.BlockSpec((BQ, 4 * DH), lambda h, i: (i, h)),
        out_shape=jax.ShapeDtypeStruct((S, D), bf16),
    )(q, k, v)

    x2, h2, logits = pl.pallas_call(
        _post_body,
        grid=(S // BM,),
        in_specs=[tok, full(D, D), full(1, D), tok, full(1, D), full(1, D),
                  full(D, E)],
        out_specs=[tok, tok, pl.BlockSpec((BM, E), lambda i: (i, 0))],
        out_shape=[jax.ShapeDtypeStruct((S, D), f32),
                   jax.ShapeDtypeStruct((S, D), f32),
                   jax.ShapeDtypeStruct((S, E), f32)],
    )(attn, Wo.astype(bf16), bo_, xs, g2, b2_, w_gate)

    gw0, gw1, d0, d1, bexp, bidx, nact = pl.pallas_call(
        _route_body,
        grid=(1,),
        in_specs=[pl.BlockSpec((S, E), lambda i: (0, 0))],
        out_specs=[pl.BlockSpec((S, 1), lambda i: (0, 0)),
                   pl.BlockSpec((S, 1), lambda i: (0, 0)),
                   pl.BlockSpec((S, 1), lambda i: (0, 0)),
                   pl.BlockSpec((S, 1), lambda i: (0, 0)),
                   pl.BlockSpec((1, 128), lambda i: (0, 0)),
                   pl.BlockSpec((1, 128), lambda i: (0, 0)),
                   pl.BlockSpec((1, 1), lambda i: (0, 0))],
        out_shape=[jax.ShapeDtypeStruct((S, 1), f32),
                   jax.ShapeDtypeStruct((S, 1), f32),
                   jax.ShapeDtypeStruct((S, 1), jnp.int32),
                   jax.ShapeDtypeStruct((S, 1), jnp.int32),
                   jax.ShapeDtypeStruct((1, 128), jnp.int32),
                   jax.ShapeDtypeStruct((1, 128), jnp.int32),
                   jax.ShapeDtypeStruct((1, 1), jnp.int32)],
    )(logits)

    d0f = d0.reshape(S)
    d1f = d1.reshape(S)
    xdisp = _sc_dispatch_call(h2, d0f, d1f)

    eout = pl.pallas_call(
        _gmm_body,
        grid_spec=pltpu.PrefetchScalarGridSpec(
            num_scalar_prefetch=3,
            grid=(NBLK,),
            in_specs=[
                pl.BlockSpec((BMG, D), lambda b, be, bi, na: (bi[b], 0)),
                pl.BlockSpec((1, D, HID),
                             lambda b, be, bi, na: (be[b], 0, 0)),
                pl.BlockSpec((1, 1, HID),
                             lambda b, be, bi, na: (be[b], 0, 0)),
                pl.BlockSpec((1, HID, D),
                             lambda b, be, bi, na: (be[b], 0, 0)),
                pl.BlockSpec((1, 1, D),
                             lambda b, be, bi, na: (be[b], 0, 0)),
            ],
            out_specs=pl.BlockSpec((BMG, D), lambda b, be, bi, na: (bi[b], 0)),
        ),
        out_shape=jax.ShapeDtypeStruct((P, D), f32),
    )(bexp.reshape(128), bidx.reshape(128), nact.reshape(1), xdisp,
      W1, b1.reshape(E, 1, HID), W2, b2.reshape(E, 1, D))

    e0, e1 = _sc_gather_call(eout, d0f, d1f)

    y = pl.pallas_call(
        _combine_body,
        grid=(S // BM,),
        in_specs=[tok, tok,
                  pl.BlockSpec((BM, 1), lambda i: (i, 0)),
                  pl.BlockSpec((BM, 1), lambda i: (i, 0)),
                  tok],
        out_specs=tok,
        out_shape=jax.ShapeDtypeStruct((S, D), f32),
    )(e0, e1, gw0, gw1, x2)

    return y.reshape(1, S, D)


# speedup vs baseline: 1.0024x; 1.0024x over previous
import functools
import math
import jax
import jax.numpy as jnp
from jax import lax
from jax.experimental import pallas as pl
from jax.experimental.pallas import tpu as pltpu
from jax.experimental.pallas import tpu_sc as plsc

S, D, H, E, K, HID = 2048, 768, 12, 8, 2, 3072
DH = D // H
BM = 512
BQ = 512
BMG = 256
P = S * K + E * BMG
NBLK = P // BMG
NC = 2
NS = 16
NW = NC * NS
TOK_W = S // NW


def _ln(x, g, b):
    m = jnp.mean(x, axis=-1, keepdims=True)
    xc = x - m
    v = jnp.mean(xc * xc, axis=-1, keepdims=True)
    return xc * jax.lax.rsqrt(v + 1e-6) * g + b


def _bf(x):
    return x.astype(jnp.bfloat16)


def _ln_qkv_body(x_ref, g_ref, b_ref, wq_ref, wk_ref, wv_ref,
                 q_ref, k_ref, v_ref):
    h = _bf(_ln(x_ref[...], g_ref[...], b_ref[...]))
    q = jnp.dot(h, wq_ref[...], preferred_element_type=jnp.float32)
    q_ref[...] = _bf(q * (1.0 / math.sqrt(DH)))
    k_ref[...] = _bf(jnp.dot(h, wk_ref[...],
                             preferred_element_type=jnp.float32))
    v_ref[...] = _bf(jnp.dot(h, wv_ref[...],
                             preferred_element_type=jnp.float32))


def _attn_body(q_ref, k_ref, v_ref, o_ref):
    q2 = q_ref[...]
    k2 = k_ref[...]
    v2 = v_ref[...]
    for j in range(4):
        q = q2[:, j * DH:(j + 1) * DH]
        k = k2[:, j * DH:(j + 1) * DH]
        v = v2[:, j * DH:(j + 1) * DH]
        s = jax.lax.dot_general(q, k, (((1,), (1,)), ((), ())),
                                preferred_element_type=jnp.float32)
        p = jnp.exp(s)
        den = jnp.sum(p, axis=-1, keepdims=True)
        o = jnp.dot(_bf(p), v, preferred_element_type=jnp.float32)
        o_ref[:, j * DH:(j + 1) * DH] = _bf(o / den)


def _post_body(a_ref, wo_ref, bo_ref, x_ref, g_ref, b_ref, wg_ref,
               x2_ref, h2_ref, lg_ref):
    o = jnp.dot(a_ref[...], wo_ref[...],
                preferred_element_type=jnp.float32)
    x2 = o + bo_ref[...] + x_ref[...]
    x2_ref[...] = x2
    h2 = _ln(x2, g_ref[...], b_ref[...])
    h2_ref[...] = h2
    lg_ref[...] = jnp.dot(h2, wg_ref[...], preferred_element_type=jnp.float32)


def _route_body(lg_ref, gw0_ref, gw1_ref, d0_ref, d1_ref, be_ref, bi_ref,
                na_ref):
    logits = lg_ref[...]
    iota = lax.broadcasted_iota(jnp.int32, (S, E), 1)
    v1 = jnp.max(logits, -1, keepdims=True)
    i1 = jnp.min(jnp.where(logits == v1, iota, E), -1, keepdims=True)
    l2 = jnp.where(iota == i1, -jnp.inf, logits)
    v2 = jnp.max(l2, -1, keepdims=True)
    i2 = jnp.min(jnp.where(l2 == v2, iota, E), -1, keepdims=True)
    r = jnp.exp(v2 - v1)
    gw0_ref[...] = 1.0 / (1.0 + r)
    gw1_ref[...] = r / (1.0 + r)

    oh0 = (iota == i1).astype(jnp.float32)
    oh1 = (iota == i2).astype(jnp.float32)
    soh = oh0 + oh1
    CH = 128
    a_io = lax.broadcasted_iota(jnp.int32, (CH, CH), 0)
    b_io = lax.broadcasted_iota(jnp.int32, (CH, CH), 1)
    tril = (a_io > b_io).astype(jnp.float32)
    run = jnp.zeros((1, E), jnp.float32)
    parts = []
    for c in range(S // CH):
        ch = soh[c * CH:(c + 1) * CH, :]
        parts.append(jnp.dot(tril, ch, preferred_element_type=jnp.float32)
                     + run)
        run = run + jnp.sum(ch, axis=0, keepdims=True)
    excl = jnp.concatenate(parts, axis=0)
    counts = run
    padded = jnp.ceil(counts / BMG) * BMG
    eu = lax.broadcasted_iota(jnp.int32, (E, E), 0)
    ev = lax.broadcasted_iota(jnp.int32, (E, E), 1)
    sut = (eu < ev).astype(jnp.float32)
    pstart = jnp.dot(padded, sut,
                     preferred_element_type=jnp.float32)
    pos = pstart + excl
    d0_ref[...] = jnp.sum(oh0 * pos, 1, keepdims=True).astype(jnp.int32)
    d1_ref[...] = jnp.sum(oh1 * pos, 1, keepdims=True).astype(jnp.int32)

    total = jnp.sum(padded, 1, keepdims=True)
    na = (total / BMG).astype(jnp.int32)
    na_ref[...] = na
    biota = lax.broadcasted_iota(jnp.int32, (1, 128), 1)
    bi_ref[...] = jnp.minimum(biota, na - 1)
    bstart = (biota * BMG).astype(jnp.float32)
    bexp = jnp.zeros((1, 128), jnp.int32)
    last_e = jnp.zeros((1, 1), jnp.float32)
    for e in range(E):
        ps_e = pstart[:, e:e + 1]
        pd_e = padded[:, e:e + 1]
        cond = (bstart >= ps_e) & (bstart < ps_e + pd_e)
        bexp = bexp + jnp.where(cond, e, 0)
        last_e = jnp.where(pd_e > 0, float(e), last_e)
    bexp = jnp.where(bstart >= total, last_e.astype(jnp.int32), bexp)
    be_ref[...] = bexp


def _gmm_body(be_ref, bi_ref, na_ref, xs_ref, w1_ref, b1_ref, w2_ref, b2_ref,
              out_ref):
    b = pl.program_id(0)

    @pl.when(b < na_ref[0])
    def _():
        mid = jnp.dot(_bf(xs_ref[...]), _bf(w1_ref[0]),
                      preferred_element_type=jnp.float32) + b1_ref[0]
        mid = _bf(jax.nn.gelu(mid))
        out_ref[...] = jnp.dot(mid, _bf(w2_ref[0]),
                               preferred_element_type=jnp.float32) + b2_ref[0]


def _combine_body(e0_ref, e1_ref, gw0_ref, gw1_ref, x2_ref, y_ref):
    y_ref[...] = (gw0_ref[...] * e0_ref[...] + gw1_ref[...] * e1_ref[...]
                  + x2_ref[...])


def _sc_dispatch_call(h2, d0, d1):
    mesh = plsc.VectorSubcoreMesh(core_axis_name="c", subcore_axis_name="s")

    @functools.partial(
        pl.kernel, mesh=mesh,
        out_type=jax.ShapeDtypeStruct((P, D), jnp.float32),
        scratch_types=[
            pltpu.VMEM((TOK_W,), jnp.int32),
            pltpu.VMEM((TOK_W,), jnp.int32),
            pltpu.VMEM((TOK_W, D), jnp.float32),
            pltpu.SemaphoreType.DMA,
            pltpu.SemaphoreType.DMA,
        ],
    )
    def k(h2_hbm, d0_hbm, d1_hbm, xs_hbm, i0_v, i1_v, rows_v, sem0, sem1):
        wid = lax.axis_index("s") * NC + lax.axis_index("c")
        base = wid * TOK_W
        pltpu.sync_copy(d0_hbm.at[pl.ds(base, TOK_W)], i0_v)
        pltpu.sync_copy(d1_hbm.at[pl.ds(base, TOK_W)], i1_v)
        pltpu.sync_copy(h2_hbm.at[pl.ds(base, TOK_W)], rows_v)
        c0 = pltpu.async_copy(rows_v, xs_hbm.at[i0_v], sem0)
        c1 = pltpu.async_copy(rows_v, xs_hbm.at[i1_v], sem1)
        c0.wait()
        c1.wait()

    return k(h2, d0, d1)


def _sc_gather_call(eout, d0, d1):
    mesh = plsc.VectorSubcoreMesh(core_axis_name="c", subcore_axis_name="s")

    @functools.partial(
        pl.kernel, mesh=mesh,
        out_type=[jax.ShapeDtypeStruct((S, D), jnp.float32),
                  jax.ShapeDtypeStruct((S, D), jnp.float32)],
        scratch_types=[
            pltpu.VMEM((TOK_W,), jnp.int32),
            pltpu.VMEM((TOK_W,), jnp.int32),
            pltpu.VMEM((TOK_W, D), jnp.float32),
            pltpu.VMEM((TOK_W, D), jnp.float32),
            pltpu.SemaphoreType.DMA,
            pltpu.SemaphoreType.DMA,
        ],
    )
    def k(eo_hbm, d0_hbm, d1_hbm, e0_hbm, e1_hbm,
          i0_v, i1_v, r0_v, r1_v, sem0, sem1):
        wid = lax.axis_index("s") * NC + lax.axis_index("c")
        base = wid * TOK_W
        pltpu.sync_copy(d0_hbm.at[pl.ds(base, TOK_W)], i0_v)
        pltpu.sync_copy(d1_hbm.at[pl.ds(base, TOK_W)], i1_v)
        c0 = pltpu.async_copy(eo_hbm.at[i0_v], r0_v, sem0)
        c1 = pltpu.async_copy(eo_hbm.at[i1_v], r1_v, sem1)
        c0.wait()
        c1.wait()
        pltpu.sync_copy(r0_v, e0_hbm.at[pl.ds(base, TOK_W)])
        pltpu.sync_copy(r1_v, e1_hbm.at[pl.ds(base, TOK_W)])

    return k(eout, d0, d1)


def kernel(x, Wq, Wk, Wv, Wo, bo, ln1_g, ln1_b, ln2_g, ln2_b,
           w_gate, W1, b1, W2, b2):
    xs = x.reshape(S, D)
    g1 = ln1_g.reshape(1, D)
    b1_ = ln1_b.reshape(1, D)
    g2 = ln2_g.reshape(1, D)
    b2_ = ln2_b.reshape(1, D)
    bo_ = bo.reshape(1, D)

    full = lambda *shape: pl.BlockSpec(shape, lambda *_: (0,) * len(shape))
    tok = pl.BlockSpec((BM, D), lambda i: (i, 0))
    f32 = jnp.float32
    bf16 = jnp.bfloat16

    q, k, v = pl.pallas_call(
        _ln_qkv_body,
        grid=(S // BM,),
        in_specs=[tok, full(1, D), full(1, D),
                  full(D, D), full(D, D), full(D, D)],
        out_specs=[tok, tok, tok],
        out_shape=[jax.ShapeDtypeStruct((S, D), bf16)] * 3,
    )(xs, g1, b1_, Wq.astype(bf16), Wk.astype(bf16), Wv.astype(bf16))

    attn = pl.pallas_call(
        _attn_body,
        grid=(H // 4, S // BQ),
        in_specs=[pl.BlockSpec((BQ, 4 * DH), lambda h, i: (i, h)),
                  pl.BlockSpec((S, 4 * DH), lambda h, i: (0, h)),
                  pl.BlockSpec((S, 4 * DH), lambda h, i: (0, h))],
        out_specs=pl.BlockSpec((BQ, 4 * DH), lambda h, i: (i, h)),
        out_shape=jax.ShapeDtypeStruct((S, D), bf16),
    )(q, k, v)

    x2, h2, logits = pl.pallas_call(
        _post_body,
        grid=(S // BM,),
        in_specs=[tok, full(D, D), full(1, D), tok, full(1, D), full(1, D),
                  full(D, E)],
        out_specs=[tok, tok, pl.BlockSpec((BM, E), lambda i: (i, 0))],
        out_shape=[jax.ShapeDtypeStruct((S, D), f32),
                   jax.ShapeDtypeStruct((S, D), f32),
                   jax.ShapeDtypeStruct((S, E), f32)],
    )(attn, Wo.astype(bf16), bo_, xs, g2, b2_, w_gate)

    gw0, gw1, d0, d1, bexp, bidx, nact = pl.pallas_call(
        _route_body,
        grid=(1,),
        in_specs=[pl.BlockSpec((S, E), lambda i: (0, 0))],
        out_specs=[pl.BlockSpec((S, 1), lambda i: (0, 0)),
                   pl.BlockSpec((S, 1), lambda i: (0, 0)),
                   pl.BlockSpec((S, 1), lambda i: (0, 0)),
                   pl.BlockSpec((S, 1), lambda i: (0, 0)),
                   pl.BlockSpec((1, 128), lambda i: (0, 0)),
                   pl.BlockSpec((1, 128), lambda i: (0, 0)),
                   pl.BlockSpec((1, 1), lambda i: (0, 0))],
        out_shape=[jax.ShapeDtypeStruct((S, 1), f32),
                   jax.ShapeDtypeStruct((S, 1), f32),
                   jax.ShapeDtypeStruct((S, 1), jnp.int32),
                   jax.ShapeDtypeStruct((S, 1), jnp.int32),
                   jax.ShapeDtypeStruct((1, 128), jnp.int32),
                   jax.ShapeDtypeStruct((1, 128), jnp.int32),
                   jax.ShapeDtypeStruct((1, 1), jnp.int32)],
    )(logits)

    d0f = d0.reshape(S)
    d1f = d1.reshape(S)
    xdisp = _sc_dispatch_call(h2, d0f, d1f)

    eout = pl.pallas_call(
        _gmm_body,
        grid_spec=pltpu.PrefetchScalarGridSpec(
            num_scalar_prefetch=3,
            grid=(NBLK,),
            in_specs=[
                pl.BlockSpec((BMG, D), lambda b, be, bi, na: (bi[b], 0)),
                pl.BlockSpec((1, D, HID),
                             lambda b, be, bi, na: (be[b], 0, 0)),
                pl.BlockSpec((1, 1, HID),
                             lambda b, be, bi, na: (be[b], 0, 0)),
                pl.BlockSpec((1, HID, D),
                             lambda b, be, bi, na: (be[b], 0, 0)),
                pl.BlockSpec((1, 1, D),
                             lambda b, be, bi, na: (be[b], 0, 0)),
            ],
            out_specs=pl.BlockSpec((BMG, D), lambda b, be, bi, na: (bi[b], 0)),
        ),
        out_shape=jax.ShapeDtypeStruct((P, D), f32),
    )(bexp.reshape(128), bidx.reshape(128), nact.reshape(1), xdisp,
      W1, b1.reshape(E, 1, HID), W2, b2.reshape(E, 1, D))

    e0, e1 = _sc_gather_call(eout, d0f, d1f)

    y = pl.pallas_call(
        _combine_body,
        grid=(S // BM,),
        in_specs=[tok, tok,
                  pl.BlockSpec((BM, 1), lambda i: (i, 0)),
                  pl.BlockSpec((BM, 1), lambda i: (i, 0)),
                  tok],
        out_specs=tok,
        out_shape=jax.ShapeDtypeStruct((S, D), f32),
    )(e0, e1, gw0, gw1, x2)

    return y.reshape(1, S, D)
